# baseline (speedup 1.0000x reference)
import functools

import jax
import jax.numpy as jnp
from jax.experimental import pallas as pl

N = 10000
BM = 400


def _s1_kernel(x_ref, w1_ref, o_ref):
    o_ref[...] = jnp.dot(x_ref[...], w1_ref[...],
                         preferred_element_type=jnp.float32)


def _layer1_kernel(adj_ref, s1_ref, b1_ref, w2_ref, t_ref):
    acc = jnp.dot(adj_ref[...], s1_ref[...],
                  preferred_element_type=jnp.float32)
    h = jnp.maximum(acc + b1_ref[...], 0.0)
    t_ref[...] = jnp.dot(h, w2_ref[...], preferred_element_type=jnp.float32)


def _layer2_kernel(adj_ref, t_ref, b2_ref, o_ref):
    o = jnp.dot(adj_ref[...], t_ref[...],
                preferred_element_type=jnp.float32) + b2_ref[...]
    m = jnp.max(o, axis=1, keepdims=True)
    lse = m + jnp.log(jnp.sum(jnp.exp(o - m), axis=1, keepdims=True))
    o_ref[...] = o - lse


@jax.jit
def kernel(x, adj, W1, b1, W2, b2):
    nfeat = x.shape[1]
    nhid = W1.shape[1]
    ncls = W2.shape[1]
    b1 = b1.reshape(1, nhid)
    b2 = b2.reshape(1, ncls)

    s1 = pl.pallas_call(
        _s1_kernel,
        grid=(5,),
        in_specs=[
            pl.BlockSpec((N // 5, nfeat), lambda i: (i, 0)),
            pl.BlockSpec((nfeat, nhid), lambda i: (0, 0)),
        ],
        out_specs=pl.BlockSpec((N // 5, nhid), lambda i: (i, 0)),
        out_shape=jax.ShapeDtypeStruct((N, nhid), jnp.float32),
    )(x, W1)

    t = pl.pallas_call(
        _layer1_kernel,
        grid=(N // BM,),
        in_specs=[
            pl.BlockSpec((BM, N), lambda i: (i, 0)),
            pl.BlockSpec((N, nhid), lambda i: (0, 0)),
            pl.BlockSpec((1, nhid), lambda i: (0, 0)),
            pl.BlockSpec((nhid, ncls), lambda i: (0, 0)),
        ],
        out_specs=pl.BlockSpec((BM, ncls), lambda i: (i, 0)),
        out_shape=jax.ShapeDtypeStruct((N, ncls), jnp.float32),
    )(adj, s1, b1, W2)

    out = pl.pallas_call(
        _layer2_kernel,
        grid=(N // BM,),
        in_specs=[
            pl.BlockSpec((BM, N), lambda i: (i, 0)),
            pl.BlockSpec((N, ncls), lambda i: (0, 0)),
            pl.BlockSpec((1, ncls), lambda i: (0, 0)),
        ],
        out_specs=pl.BlockSpec((BM, ncls), lambda i: (i, 0)),
        out_shape=jax.ShapeDtypeStruct((N, ncls), jnp.float32),
    )(adj, t, b2)
    return out


# speedup vs baseline: 1.0537x; 1.0537x over previous
import jax
import jax.numpy as jnp
from jax.experimental import pallas as pl
from jax.experimental.pallas import tpu as pltpu

N = 10000
BM = 400


def _mega_kernel(x_ref, adj_ref, w1_ref, b1_ref, w2_ref, b2_ref, out_ref,
                 s1_ref, t_ref):
    p = pl.program_id(0)
    i = pl.program_id(1)

    @pl.when((p == 0) & (i == 0))
    def _():
        s1_ref[...] = jnp.dot(x_ref[...], w1_ref[...],
                              preferred_element_type=jnp.float32)

    @pl.when(p == 0)
    def _():
        acc = jnp.dot(adj_ref[...], s1_ref[...],
                      preferred_element_type=jnp.float32)
        h = jnp.maximum(acc + b1_ref[...], 0.0)
        t_ref[pl.ds(i * BM, BM), :] = jnp.dot(
            h, w2_ref[...], preferred_element_type=jnp.float32)

    @pl.when(p == 1)
    def _():
        o = jnp.dot(adj_ref[...], t_ref[...],
                    preferred_element_type=jnp.float32) + b2_ref[...]
        m = jnp.max(o, axis=1, keepdims=True)
        lse = m + jnp.log(jnp.sum(jnp.exp(o - m), axis=1, keepdims=True))
        out_ref[...] = o - lse


@jax.jit
def kernel(x, adj, W1, b1, W2, b2):
    nfeat = x.shape[1]
    nhid = W1.shape[1]
    ncls = W2.shape[1]
    b1 = b1.reshape(1, nhid)
    b2 = b2.reshape(1, ncls)

    return pl.pallas_call(
        _mega_kernel,
        grid=(2, N // BM),
        in_specs=[
            pl.BlockSpec((N, nfeat), lambda p, i: (0, 0)),
            pl.BlockSpec((BM, N), lambda p, i: (i, 0)),
            pl.BlockSpec((nfeat, nhid), lambda p, i: (0, 0)),
            pl.BlockSpec((1, nhid), lambda p, i: (0, 0)),
            pl.BlockSpec((nhid, ncls), lambda p, i: (0, 0)),
            pl.BlockSpec((1, ncls), lambda p, i: (0, 0)),
        ],
        out_specs=pl.BlockSpec((BM, ncls), lambda p, i: (i, 0)),
        out_shape=jax.ShapeDtypeStruct((N, ncls), jnp.float32),
        scratch_shapes=[
            pltpu.VMEM((N, nhid), jnp.float32),
            pltpu.VMEM((N, ncls), jnp.float32),
        ],
    )(x, adj, W1, b1, W2, b2)


# speedup vs baseline: 1.3454x; 1.2768x over previous
import numpy as np

import jax
import jax.numpy as jnp
from jax import lax
from jax.experimental import pallas as pl
from jax.experimental.pallas import tpu as pltpu

N = 10000
BT = 2048
P = -(-N // BT)
NPAD = P * BT

_SCHED = []
for _r in range(P):
    for _c in [c for c in range(P) if c != _r] + [_r]:
        _SCHED.append((_r, _c))
for _r in range(P):
    for _c in range(_r + 1, P):
        _SCHED.append((_r, _c))
_STEPS = len(_SCHED)
_RS = np.array([rc[0] for rc in _SCHED], np.int32)
_CS = np.array([rc[1] for rc in _SCHED], np.int32)
_COMP = {r: max(s for s, rc in enumerate(_SCHED) if rc[0] == r)
         for r in range(P)}
_WBLK = np.zeros(_STEPS, np.int32)
_WFLAG = np.zeros(_STEPS, np.int32)
_prev = -1
for _row in sorted(range(P), key=lambda r: _COMP[r]):
    _s = _COMP[_row]
    _WBLK[_prev + 1:_s + 1] = _row
    _WFLAG[_s] = 1
    _prev = _s


def _tile_kernel(rs_ref, cs_ref, wb_ref, wf_ref,
                 x_ref, adj_ref, w1_ref, b1_ref, w2_ref, b2_ref,
                 out_ref, s1_ref, t_ref, acc1_ref, acc2_ref):
    s = pl.program_id(0)
    r = rs_ref[s]
    c = cs_ref[s]
    in_a = s < P * P
    first_in_row = in_a & (s % P == 0)

    @pl.when(s == 0)
    def _():
        s1_ref[pl.ds(0, N), :] = jnp.dot(x_ref[...], w1_ref[...],
                                         preferred_element_type=jnp.float32)
        s1_ref[pl.ds(N, NPAD - N), :] = jnp.zeros(
            (NPAD - N, s1_ref.shape[1]), jnp.float32)
        acc2_ref[...] = jnp.zeros_like(acc2_ref)

    @pl.when(c == P - 1)
    def _():
        adj_ref[:, pl.ds(N - (P - 1) * BT, NPAD - N)] = jnp.zeros(
            (BT, NPAD - N), jnp.float32)

    @pl.when(in_a)
    def _():
        p1 = jnp.dot(adj_ref[...], s1_ref[pl.ds(c * BT, BT), :],
                     preferred_element_type=jnp.float32)

        @pl.when(first_in_row)
        def _():
            acc1_ref[...] = p1

        @pl.when(jnp.logical_not(first_in_row))
        def _():
            acc1_ref[...] = acc1_ref[...] + p1

    @pl.when(in_a & (r == c))
    def _():
        h = jnp.maximum(acc1_ref[...] + b1_ref[...], 0.0)
        tt = jnp.dot(h, w2_ref[...], preferred_element_type=jnp.float32)
        row_ok = (lax.broadcasted_iota(jnp.int32, tt.shape, 0)
                  + r * BT) < N
        t_ref[pl.ds(r * BT, BT), :] = jnp.where(row_ok, tt, 0.0)

    @pl.when(jnp.logical_not(in_a) | (c <= r))
    def _():
        p2 = jnp.dot(adj_ref[...], t_ref[pl.ds(c * BT, BT), :],
                     preferred_element_type=jnp.float32)
        acc2_ref[pl.ds(r * BT, BT), :] = (
            acc2_ref[pl.ds(r * BT, BT), :] + p2)

    @pl.when(wf_ref[s] == 1)
    def _():
        w = wb_ref[s]
        o = acc2_ref[pl.ds(w * BT, BT), :] + b2_ref[...]
        m = jnp.max(o, axis=1, keepdims=True)
        lse = m + jnp.log(jnp.sum(jnp.exp(o - m), axis=1, keepdims=True))
        out_ref[...] = o - lse


@jax.jit
def kernel(x, adj, W1, b1, W2, b2):
    nfeat = x.shape[1]
    nhid = W1.shape[1]
    ncls = W2.shape[1]
    b1 = b1.reshape(1, nhid)
    b2 = b2.reshape(1, ncls)

    grid_spec = pltpu.PrefetchScalarGridSpec(
        num_scalar_prefetch=4,
        grid=(_STEPS,),
        in_specs=[
            pl.BlockSpec((N, nfeat), lambda s, rs, cs, wb, wf: (0, 0)),
            pl.BlockSpec((BT, BT), lambda s, rs, cs, wb, wf: (rs[s], cs[s])),
            pl.BlockSpec((nfeat, nhid), lambda s, rs, cs, wb, wf: (0, 0)),
            pl.BlockSpec((1, nhid), lambda s, rs, cs, wb, wf: (0, 0)),
            pl.BlockSpec((nhid, ncls), lambda s, rs, cs, wb, wf: (0, 0)),
            pl.BlockSpec((1, ncls), lambda s, rs, cs, wb, wf: (0, 0)),
        ],
        out_specs=pl.BlockSpec((BT, ncls),
                               lambda s, rs, cs, wb, wf: (wb[s], 0)),
        scratch_shapes=[
            pltpu.VMEM((NPAD, nhid), jnp.float32),
            pltpu.VMEM((NPAD, ncls), jnp.float32),
            pltpu.VMEM((BT, nhid), jnp.float32),
            pltpu.VMEM((NPAD, ncls), jnp.float32),
        ],
    )
    return pl.pallas_call(
        _tile_kernel,
        grid_spec=grid_spec,
        out_shape=jax.ShapeDtypeStruct((N, ncls), jnp.float32),
    )(jnp.asarray(_RS), jnp.asarray(_CS), jnp.asarray(_WBLK),
      jnp.asarray(_WFLAG), x, adj, W1, b1, W2, b2)
